# baseline (speedup 1.0000x reference)
import functools

import jax
import jax.numpy as jnp
from jax import lax
from jax.experimental import pallas as pl
from jax.experimental.pallas import tpu as pltpu
from jax.experimental.pallas import tpu_sc as plsc

_K = 1024
_D = 64
_BETA = 0.25
_KC = 256

_NC = 2
_NS = 16
_NW = _NC * _NS
_L = 16
_HD = 32


def _argmin_body(xt_ref, xsq_ref, w_ref, wsq_ref, kio_ref, idx_ref, loss_ref,
                 acc_ref, *, n_tokens, r):
    x2t = xt_ref[0] * 2.0
    xsq_row = xsq_ref[0]

    big = float(2 * _K)
    run_min8 = jnp.full((8, r), jnp.inf, dtype=jnp.float32)
    run_k8 = jnp.zeros((8, r), dtype=jnp.float32)
    for c in range(_K // _KC):
        wc = w_ref[c * _KC:(c + 1) * _KC, :]
        wsq_c = wsq_ref[c * _KC:(c + 1) * _KC, :]
        mm2 = lax.dot_general(wc, x2t, (((1,), (0,)), ((), ())),
                              preferred_element_type=jnp.float32)
        dist = (xsq_row + wsq_c) - mm2
        d3 = dist.reshape(_KC // 8, 8, r)
        cmin8 = jnp.min(d3, axis=0)
        kio_c = kio_ref[c * (_KC // 8):(c + 1) * (_KC // 8), :][:, :, None]
        ck8 = jnp.min(jnp.where(d3 == cmin8[None], kio_c, big), axis=0)
        better = cmin8 < run_min8
        run_k8 = jnp.where(better, ck8, run_k8)
        run_min8 = jnp.where(better, cmin8, run_min8)

    gmin = jnp.min(run_min8, axis=0)
    kbest = jnp.min(jnp.where(run_min8 == gmin[None, :], run_k8, big), axis=0)
    idx_ref[...] = kbest.astype(jnp.int32)[None, None, :]

    @pl.when(pl.program_id(0) == 0)
    def _init():
        acc_ref[0, 0] = 0.0

    acc_ref[0, 0] += jnp.sum(gmin)
    scale = (1.0 + _BETA) / (n_tokens * _D)
    loss_ref[...] = jnp.broadcast_to(acc_ref[0, 0] * scale, (1, 1))


def _lookup_body(w_hbm, idx_hbm, out_hbm, w_v, idx_v, half, wsems,
                 *, nb, node):
    wid = lax.axis_index("s") * _NC + lax.axis_index("c")
    bpw = nb // _NW
    pltpu.sync_copy(w_hbm, w_v)
    wcps = [None, None]
    for bb in range(bpw):
        b = wid * bpw + bb
        pltpu.sync_copy(idx_hbm.at[b], idx_v)
        for h in range(2):
            if wcps[h] is not None:
                wcps[h].wait()
            buf = half[h]

            def fill(g, _, h=h, buf=buf):
                iv = idx_v[0, pl.ds(g * _L, _L)]
                for d in range(_HD):
                    col = jnp.full((_L,), h * _HD + d, dtype=jnp.int32)
                    buf[d, pl.ds(g * _L, _L)] = plsc.load_gather(
                        w_v, [iv, col])
                return _

            lax.fori_loop(0, node // _L, fill, 0)
            wcps[h] = pltpu.async_copy(
                buf, out_hbm.at[b, pl.ds(h * _HD, _HD)], wsems[h])
    for h in range(2):
        if wcps[h] is not None:
            wcps[h].wait()


def kernel(latents, W):
    lat = latents.reshape(-1, latents.shape[-2], _D)
    flat = lat.reshape(-1, _D)
    n = flat.shape[0]
    nb, node = lat.shape[0], lat.shape[1]
    latT = jnp.transpose(lat, (0, 2, 1))
    xsq = jnp.sum(flat ** 2, axis=1).reshape(nb, 1, node)
    wsq = jnp.sum(W ** 2, axis=1)[:, None]
    kio = jnp.arange(_K, dtype=jnp.float32).reshape(_K // 8, 8)

    grid = (nb,)
    idx, loss = pl.pallas_call(
        functools.partial(_argmin_body, n_tokens=n, r=node),
        grid=grid,
        in_specs=[
            pl.BlockSpec((1, _D, node), lambda i: (i, 0, 0)),
            pl.BlockSpec((1, 1, node), lambda i: (i, 0, 0)),
            pl.BlockSpec((_K, _D), lambda i: (0, 0)),
            pl.BlockSpec((_K, 1), lambda i: (0, 0)),
            pl.BlockSpec((_K // 8, 8), lambda i: (0, 0)),
        ],
        out_specs=[
            pl.BlockSpec((1, 1, node), lambda i: (i, 0, 0)),
            pl.BlockSpec((1, 1), lambda i: (0, 0)),
        ],
        out_shape=[
            jax.ShapeDtypeStruct((nb, 1, node), jnp.int32),
            jax.ShapeDtypeStruct((1, 1), jnp.float32),
        ],
        scratch_shapes=[pltpu.SMEM((1, 1), jnp.float32)],
    )(latT, xsq, W, wsq, kio)

    mesh = plsc.VectorSubcoreMesh(core_axis_name="c", subcore_axis_name="s")
    qt = pl.kernel(
        functools.partial(_lookup_body, nb=nb, node=node),
        mesh=mesh,
        out_type=jax.ShapeDtypeStruct((nb, _D, node), jnp.float32),
        scratch_types=[
            pltpu.VMEM((_K, _D), jnp.float32),
            pltpu.VMEM((1, node), jnp.int32),
            [pltpu.VMEM((_HD, node), jnp.float32) for _ in range(2)],
            [pltpu.SemaphoreType.DMA for _ in range(2)],
        ],
        compiler_params=pltpu.CompilerParams(use_tc_tiling_on_sc=False,
                                             needs_layout_passes=False),
    )(W, idx)

    quantized_st = qt.transpose(0, 2, 1).reshape(lat.shape)
    vq_loss = loss[0, 0]
    return (quantized_st, vq_loss)


# speedup vs baseline: 1.2584x; 1.2584x over previous
import functools

import jax
import jax.numpy as jnp
from jax import lax
from jax.experimental import pallas as pl
from jax.experimental.pallas import tpu as pltpu
from jax.experimental.pallas import tpu_sc as plsc

_K = 1024
_D = 64
_BETA = 0.25
_KC = 256

_NC = 2
_NS = 16
_NW = _NC * _NS
_CH = 128
_NBUF = 4


def _argmin_body(xt_ref, xsq_ref, w_ref, wsq_ref, kio_ref, idx_ref, loss_ref,
                 acc_ref, *, n_tokens, r):
    x2t = xt_ref[0] * 2.0
    xsq_row = xsq_ref[0]

    big = float(2 * _K)
    run_min8 = jnp.full((8, r), jnp.inf, dtype=jnp.float32)
    run_k8 = jnp.zeros((8, r), dtype=jnp.float32)
    for c in range(_K // _KC):
        wc = w_ref[c * _KC:(c + 1) * _KC, :]
        wsq_c = wsq_ref[c * _KC:(c + 1) * _KC, :]
        mm2 = lax.dot_general(wc, x2t, (((1,), (0,)), ((), ())),
                              preferred_element_type=jnp.float32)
        dist = (xsq_row + wsq_c) - mm2
        d3 = dist.reshape(_KC // 8, 8, r)
        cmin8 = jnp.min(d3, axis=0)
        kio_c = kio_ref[c * (_KC // 8):(c + 1) * (_KC // 8), :][:, :, None]
        ck8 = jnp.min(jnp.where(d3 == cmin8[None], kio_c, big), axis=0)
        better = cmin8 < run_min8
        run_k8 = jnp.where(better, ck8, run_k8)
        run_min8 = jnp.where(better, cmin8, run_min8)

    gmin = jnp.min(run_min8, axis=0)
    kbest = jnp.min(jnp.where(run_min8 == gmin[None, :], run_k8, big), axis=0)
    idx_ref[...] = kbest.astype(jnp.int32)[None, None, :]

    @pl.when(pl.program_id(0) == 0)
    def _init():
        acc_ref[0, 0] = 0.0

    acc_ref[0, 0] += jnp.sum(gmin)
    scale = (1.0 + _BETA) / (n_tokens * _D)
    loss_ref[...] = jnp.broadcast_to(acc_ref[0, 0] * scale, (1, 1))


def _gather_body(w_hbm, idx_hbm, out_hbm, idx_v, rows, gsems, wsems, *, chunks):
    wid = lax.axis_index("s") * _NC + lax.axis_index("c")
    base = wid * (chunks * _CH)
    pltpu.sync_copy(idx_hbm.at[pl.ds(base, chunks * _CH)], idx_v)
    gcps = [None] * _NBUF
    wcps = [None] * _NBUF
    for j in range(chunks):
        b = j % _NBUF
        if wcps[b] is not None:
            wcps[b].wait()
        gcps[b] = pltpu.async_copy(
            w_hbm.at[idx_v.at[pl.ds(j * _CH, _CH)]], rows[b], gsems[b])
        if j >= _NBUF - 1:
            jw = j - (_NBUF - 1)
            bw = jw % _NBUF
            gcps[bw].wait()
            wcps[bw] = pltpu.async_copy(
                rows[bw], out_hbm.at[pl.ds(base + jw * _CH, _CH)], wsems[bw])
    for jw in range(max(chunks - _NBUF + 1, 0), chunks):
        bw = jw % _NBUF
        gcps[bw].wait()
        wcps[bw] = pltpu.async_copy(
            rows[bw], out_hbm.at[pl.ds(base + jw * _CH, _CH)], wsems[bw])
    for b in range(_NBUF):
        if wcps[b] is not None:
            wcps[b].wait()


def kernel(latents, W):
    lat = latents.reshape(-1, latents.shape[-2], _D)
    flat = lat.reshape(-1, _D)
    n = flat.shape[0]
    nb, node = lat.shape[0], lat.shape[1]
    latT = jnp.transpose(lat, (0, 2, 1))
    xsq = jnp.sum(flat ** 2, axis=1).reshape(nb, 1, node)
    wsq = jnp.sum(W ** 2, axis=1)[:, None]
    kio = jnp.arange(_K, dtype=jnp.float32).reshape(_K // 8, 8)

    grid = (nb,)
    idx, loss = pl.pallas_call(
        functools.partial(_argmin_body, n_tokens=n, r=node),
        grid=grid,
        in_specs=[
            pl.BlockSpec((1, _D, node), lambda i: (i, 0, 0)),
            pl.BlockSpec((1, 1, node), lambda i: (i, 0, 0)),
            pl.BlockSpec((_K, _D), lambda i: (0, 0)),
            pl.BlockSpec((_K, 1), lambda i: (0, 0)),
            pl.BlockSpec((_K // 8, 8), lambda i: (0, 0)),
        ],
        out_specs=[
            pl.BlockSpec((1, 1, node), lambda i: (i, 0, 0)),
            pl.BlockSpec((1, 1), lambda i: (0, 0)),
        ],
        out_shape=[
            jax.ShapeDtypeStruct((nb, 1, node), jnp.int32),
            jax.ShapeDtypeStruct((1, 1), jnp.float32),
        ],
        scratch_shapes=[pltpu.SMEM((1, 1), jnp.float32)],
    )(latT, xsq, W, wsq, kio)
    idx = idx.reshape(n)

    chunks = n // (_NW * _CH)
    mesh = plsc.VectorSubcoreMesh(core_axis_name="c", subcore_axis_name="s")
    q = pl.kernel(
        functools.partial(_gather_body, chunks=chunks),
        mesh=mesh,
        out_type=jax.ShapeDtypeStruct((n, _D), jnp.float32),
        scratch_types=[
            pltpu.VMEM((chunks * _CH,), jnp.int32),
            [pltpu.VMEM((_CH, _D), jnp.float32) for _ in range(_NBUF)],
            [pltpu.SemaphoreType.DMA for _ in range(_NBUF)],
            [pltpu.SemaphoreType.DMA for _ in range(_NBUF)],
        ],
        compiler_params=pltpu.CompilerParams(use_tc_tiling_on_sc=False),
    )(W, idx)

    quantized_st = q.reshape(lat.shape)
    vq_loss = loss[0, 0]
    return (quantized_st, vq_loss)
